# Initial kernel scaffold; baseline (speedup 1.0000x reference)
import jax
import jax.numpy as jnp
from jax.experimental import pallas as pl


def kernel(x, gauges, kernel_vals, edge_index, n_id, W1, b1, W2, b2):
    raise NotImplementedError("write your pallas kernel here")


# speedup vs baseline: 8.7625x; 8.7625x over previous
import functools

import jax
import jax.numpy as jnp
from jax import lax
from jax.experimental import pallas as pl
from jax.experimental.pallas import tpu as pltpu
from jax.experimental.pallas import tpu_sc as plsc

_N = 50000
_DIM = 16
_E = 1600000
_NC = 2
_NS = 16
_NW = _NC * _NS
_LANES = 128
_SB = 8
_RPW = 392
_R = _NW * _RPW
_EP = _R * _LANES
_NSUP = _RPW // _SB
_NP = 50048
_NPT = _NP // _NS
_ZROWS = 136


def _prep_body(x_ref, g_ref, h_ref):
    x = x_ref[...]
    h = x[:, 0:1] * g_ref[:, 0, :]
    for d in range(1, _DIM):
        h = h + x[:, d : d + 1] * g_ref[:, d, :]
    nrm = jnp.maximum(jnp.sqrt(jnp.sum(h * h, axis=1, keepdims=True)), 1e-12)
    h_ref[...] = h / nrm


def _prep(x, gauges):
    B = 1000
    return pl.pallas_call(
        _prep_body,
        grid=(_N // B,),
        in_specs=[
            pl.BlockSpec((B, _DIM), lambda i: (i, 0)),
            pl.BlockSpec((B, _DIM, _DIM), lambda i: (i, 0, 0)),
        ],
        out_specs=pl.BlockSpec((B, _DIM), lambda i: (i, 0)),
        out_shape=jax.ShapeDtypeStruct((_N, _DIM), jnp.float32),
    )(x, gauges)


def _msgpass_body(table_h, src_h, dst_h, kv0_h, kv1_h, out_h,
                  acc0, acc1, srcb, dstb, kv0b, kv1b, rows, msg1, zbuf, gsem):
    c = lax.axis_index("c")
    s = lax.axis_index("s")
    wid = s * _NC + c

    def _zrow(i, _):
        zbuf[i] = jnp.zeros((_DIM,), jnp.float32)
        return 0

    lax.fori_loop(0, _ZROWS, _zrow, 0)
    base = s * _NPT

    def _zcp(i, _):
        pltpu.sync_copy(zbuf, acc0.at[pl.ds(base + i * _ZROWS, _ZROWS)])
        pltpu.sync_copy(zbuf, acc1.at[pl.ds(base + i * _ZROWS, _ZROWS)])
        return 0

    lax.fori_loop(0, _NPT // _ZROWS, _zcp, 0)
    plsc.subcore_barrier()

    row0 = wid * _RPW

    def _sup(t, _):
        r0 = row0 + t * _SB
        pltpu.sync_copy(src_h.at[pl.ds(r0, _SB)], srcb)
        pltpu.sync_copy(dst_h.at[pl.ds(r0, _SB)], dstb)
        pltpu.sync_copy(kv0_h.at[pl.ds(r0, _SB)], kv0b)
        pltpu.sync_copy(kv1_h.at[pl.ds(r0, _SB)], kv1b)
        for r in range(_SB):
            pltpu.async_copy(table_h.at[srcb.at[r]], rows, gsem).wait()

            def _blk(b, _):
                kvec0 = kv0b[r, pl.ds(b * _DIM, _DIM)]
                kvec1 = kv1b[r, pl.ds(b * _DIM, _DIM)]
                for ee in range(_DIM):
                    e = b * _DIM + ee
                    v = rows[e]
                    msg1[e] = v * kvec1[ee]
                    rows[e] = v * kvec0[ee]
                return 0

            lax.fori_loop(0, _LANES // _DIM, _blk, 0)
            pltpu.sync_copy(rows, acc0.at[dstb.at[r]], add=True)
            pltpu.sync_copy(msg1, acc1.at[dstb.at[r]], add=True)
        return 0

    lax.fori_loop(0, _NSUP, _sup, 0)
    plsc.subcore_barrier()

    pltpu.sync_copy(acc0.at[pl.ds(base, _NPT)], out_h.at[c, 0, pl.ds(base, _NPT)])
    pltpu.sync_copy(acc1.at[pl.ds(base, _NPT)], out_h.at[c, 1, pl.ds(base, _NPT)])


def _msgpass(table, src2d, dst2d, kv02d, kv12d):
    mesh = plsc.VectorSubcoreMesh(
        core_axis_name="c", subcore_axis_name="s",
        num_cores=_NC, num_subcores=_NS,
    )
    fn = functools.partial(
        pl.kernel,
        out_type=jax.ShapeDtypeStruct((_NC, 2, _NP, _DIM), jnp.float32),
        mesh=mesh,
        compiler_params=pltpu.CompilerParams(use_tc_tiling_on_sc=False),
        scratch_types=[
            pltpu.VMEM_SHARED((_NP, _DIM), jnp.float32),
            pltpu.VMEM_SHARED((_NP, _DIM), jnp.float32),
            pltpu.VMEM((_SB, _LANES), jnp.int32),
            pltpu.VMEM((_SB, _LANES), jnp.int32),
            pltpu.VMEM((_SB, _LANES), jnp.float32),
            pltpu.VMEM((_SB, _LANES), jnp.float32),
            pltpu.VMEM((_LANES, _DIM), jnp.float32),
            pltpu.VMEM((_LANES, _DIM), jnp.float32),
            pltpu.VMEM((_ZROWS, _DIM), jnp.float32),
            pltpu.SemaphoreType.DMA,
        ],
    )(_msgpass_body)
    return fn(table, src2d, dst2d, kv02d, kv12d)


def _merge_body(p_ref, c0_ref, c1_ref):
    c0_ref[...] = p_ref[0, 0] + p_ref[1, 0]
    c1_ref[...] = p_ref[0, 1] + p_ref[1, 1]


def _merge(p1):
    B = 2000
    out = jax.ShapeDtypeStruct((_N, _DIM), jnp.float32)
    return pl.pallas_call(
        _merge_body,
        grid=(_N // B,),
        in_specs=[pl.BlockSpec((_NC, 2, B, _DIM), lambda i: (0, 0, i, 0))],
        out_specs=[pl.BlockSpec((B, _DIM), lambda i: (i, 0))] * 2,
        out_shape=[out, out],
    )(p1)


def _mlp_body(h_ref, c0_ref, c1_ref, pa_ref, pb_ref,
              w1_ref, b1_ref, w2_ref, b2_ref, o_ref):
    a00 = pa_ref[0, 0] + pa_ref[1, 0]
    a01 = pb_ref[0, 0] + pb_ref[1, 0]
    a10 = pa_ref[0, 1] + pa_ref[1, 1]
    a11 = pb_ref[0, 1] + pb_ref[1, 1]
    feat = jnp.concatenate(
        [h_ref[...], c0_ref[...], c1_ref[...], a00, a01, a10, a11], axis=1)
    hid = jnp.dot(feat, w1_ref[...], preferred_element_type=jnp.float32)
    hid = jnp.maximum(hid + b1_ref[...][None, :], 0.0)
    o_ref[...] = (
        jnp.dot(hid, w2_ref[...], preferred_element_type=jnp.float32)
        + b2_ref[...][None, :])


def _mlp(h, c0, c1, p2a, p2b, W1, b1, W2, b2):
    B = 2000
    part = pl.BlockSpec((_NC, 2, B, _DIM), lambda i: (0, 0, i, 0))
    nd = pl.BlockSpec((B, _DIM), lambda i: (i, 0))
    return pl.pallas_call(
        _mlp_body,
        grid=(_N // B,),
        in_specs=[
            nd, nd, nd, part, part,
            pl.BlockSpec(W1.shape, lambda i: (0, 0)),
            pl.BlockSpec(b1.shape, lambda i: (0,)),
            pl.BlockSpec(W2.shape, lambda i: (0, 0)),
            pl.BlockSpec(b2.shape, lambda i: (0,)),
        ],
        out_specs=pl.BlockSpec((B, 32), lambda i: (i, 0)),
        out_shape=jax.ShapeDtypeStruct((_N, 32), jnp.float32),
    )(h, c0, c1, p2a, p2b, W1, b1, W2, b2)


def kernel(x, gauges, kernel_vals, edge_index, n_id, W1, b1, W2, b2):
    src = edge_index[0].astype(jnp.int32)
    dst = edge_index[1].astype(jnp.int32)
    kv0 = kernel_vals[0].astype(jnp.float32)
    kv1 = kernel_vals[1].astype(jnp.float32)
    pad = _EP - _E
    src2d = jnp.pad(src, (0, pad)).reshape(_R, _LANES)
    dst2d = jnp.pad(dst, (0, pad)).reshape(_R, _LANES)
    kv02d = jnp.pad(kv0, (0, pad)).reshape(_R, _LANES)
    kv12d = jnp.pad(kv1, (0, pad)).reshape(_R, _LANES)

    h = _prep(x, gauges)
    p1 = _msgpass(h, src2d, dst2d, kv02d, kv12d)
    c0, c1 = _merge(p1)
    p2a = _msgpass(c0, src2d, dst2d, kv02d, kv12d)
    p2b = _msgpass(c1, src2d, dst2d, kv02d, kv12d)
    return _mlp(h, c0, c1, p2a, p2b, W1, b1, W2, b2)


# speedup vs baseline: 11.0446x; 1.2604x over previous
import functools

import jax
import jax.numpy as jnp
from jax import lax
from jax.experimental import pallas as pl
from jax.experimental.pallas import tpu as pltpu
from jax.experimental.pallas import tpu_sc as plsc

_N = 50000
_DIM = 16
_E = 1600000
_NC = 2
_NS = 16
_NW = _NC * _NS
_LANES = 128
_SB = 2
_RPW = 392
_R = _NW * _RPW
_EP = _R * _LANES
_NSUP = _RPW // _SB
_NP = 50048
_NPT = _NP // _NS
_ZROWS = 136


def _prep_body(x_ref, g_ref, e_ref, r_ref, h_ref):
    xb = jnp.dot(x_ref[...], e_ref[...], preferred_element_type=jnp.float32)
    prod = xb * g_ref[...]
    h = jnp.dot(prod, r_ref[...], preferred_element_type=jnp.float32)
    nrm = jnp.maximum(jnp.sqrt(jnp.sum(h * h, axis=1, keepdims=True)), 1e-12)
    h_ref[...] = h / nrm


def _prep(x, gauges):
    B = 2000
    g2 = gauges.reshape(_N, _DIM * _DIM)
    em = jnp.kron(jnp.eye(_DIM, dtype=jnp.float32),
                  jnp.ones((1, _DIM), dtype=jnp.float32))
    rm = jnp.kron(jnp.ones((_DIM, 1), dtype=jnp.float32),
                  jnp.eye(_DIM, dtype=jnp.float32))
    return pl.pallas_call(
        _prep_body,
        grid=(_N // B,),
        in_specs=[
            pl.BlockSpec((B, _DIM), lambda i: (i, 0)),
            pl.BlockSpec((B, _DIM * _DIM), lambda i: (i, 0)),
            pl.BlockSpec(em.shape, lambda i: (0, 0)),
            pl.BlockSpec(rm.shape, lambda i: (0, 0)),
        ],
        out_specs=pl.BlockSpec((B, _DIM), lambda i: (i, 0)),
        out_shape=jax.ShapeDtypeStruct((_N, _DIM), jnp.float32),
    )(x, g2, em, rm)


def _msgpass_body(table_h, src_h, dst_h, kv0_h, kv1_h, out_h,
                  acc0, acc1, srcb, dstb, kv0b, kv1b, rows, msg, zbuf,
                  gsems, ssems):
    c = lax.axis_index("c")
    s = lax.axis_index("s")
    wid = s * _NC + c

    def _zrow(i, _):
        zbuf[i] = jnp.zeros((_DIM,), jnp.float32)
        return 0

    lax.fori_loop(0, _ZROWS, _zrow, 0)
    nbase = s * _NPT

    def _zcp(i, _):
        pltpu.sync_copy(zbuf, acc0.at[pl.ds(nbase + i * _ZROWS, _ZROWS)])
        pltpu.sync_copy(zbuf, acc1.at[pl.ds(nbase + i * _ZROWS, _ZROWS)])
        return 0

    lax.fori_loop(0, _NPT // _ZROWS, _zcp, 0)
    plsc.subcore_barrier()

    row0 = wid * _RPW

    def _loadidx(bank, sc):
        r0 = row0 + sc * _SB
        pltpu.sync_copy(src_h.at[pl.ds(r0, _SB)], srcb.at[bank])
        pltpu.sync_copy(dst_h.at[pl.ds(r0, _SB)], dstb.at[bank])
        pltpu.sync_copy(kv0_h.at[pl.ds(r0, _SB)], kv0b.at[bank])
        pltpu.sync_copy(kv1_h.at[pl.ds(r0, _SB)], kv1b.at[bank])

    def _fire_gathers(bank):
        for r in range(_SB):
            pltpu.async_copy(
                table_h.at[srcb.at[bank, r]], rows.at[bank, r], gsems[bank])

    def _drain_gathers(bank):
        for r in range(_SB):
            pltpu.make_async_copy(
                table_h.at[pl.ds(0, _LANES)], rows.at[bank, r],
                gsems[bank]).wait()

    def _drain_scatters(bank):
        for r in range(_SB):
            pltpu.make_async_copy(
                table_h.at[pl.ds(0, _LANES)], rows.at[bank, r],
                ssems[bank]).wait()
            pltpu.make_async_copy(
                table_h.at[pl.ds(0, _LANES)], msg.at[bank, r],
                ssems[bank]).wait()

    def _compute_scatter(bank):
        _drain_gathers(bank)
        for r in range(_SB):

            def _blk(b, _):
                kvec0 = kv0b[bank, r, pl.ds(b * _DIM, _DIM)]
                kvec1 = kv1b[bank, r, pl.ds(b * _DIM, _DIM)]
                for ee in range(_DIM):
                    e = b * _DIM + ee
                    v = rows[bank, r, e]
                    msg[bank, r, e] = v * kvec1[ee]
                    rows[bank, r, e] = v * kvec0[ee]
                return 0

            lax.fori_loop(0, _LANES // _DIM, _blk, 0)
            pltpu.async_copy(
                rows.at[bank, r], acc0.at[dstb.at[bank, r]], ssems[bank],
                add=True)
            pltpu.async_copy(
                msg.at[bank, r], acc1.at[dstb.at[bank, r]], ssems[bank],
                add=True)

    _loadidx(0, 0)
    _fire_gathers(0)

    def _sup(t2, _):
        sc0 = 2 * t2
        for bank in range(2):
            sc = sc0 + bank
            nb = 1 - bank

            @pl.when(sc >= 1)
            def _():
                _drain_scatters(nb)

            @pl.when(sc < _NSUP - 1)
            def _():
                _loadidx(nb, sc + 1)
                _fire_gathers(nb)

            _compute_scatter(bank)
        return 0

    lax.fori_loop(0, _NSUP // 2, _sup, 0)
    _drain_scatters(1)
    plsc.subcore_barrier()

    pltpu.sync_copy(acc0.at[pl.ds(nbase, _NPT)],
                    out_h.at[c, 0, pl.ds(nbase, _NPT)])
    pltpu.sync_copy(acc1.at[pl.ds(nbase, _NPT)],
                    out_h.at[c, 1, pl.ds(nbase, _NPT)])


def _msgpass(table, src2d, dst2d, kv02d, kv12d):
    mesh = plsc.VectorSubcoreMesh(
        core_axis_name="c", subcore_axis_name="s",
        num_cores=_NC, num_subcores=_NS,
    )
    fn = functools.partial(
        pl.kernel,
        out_type=jax.ShapeDtypeStruct((_NC, 2, _NP, _DIM), jnp.float32),
        mesh=mesh,
        compiler_params=pltpu.CompilerParams(use_tc_tiling_on_sc=False),
        scratch_types=[
            pltpu.VMEM_SHARED((_NP, _DIM), jnp.float32),
            pltpu.VMEM_SHARED((_NP, _DIM), jnp.float32),
            pltpu.VMEM((2, _SB, _LANES), jnp.int32),
            pltpu.VMEM((2, _SB, _LANES), jnp.int32),
            pltpu.VMEM((2, _SB, _LANES), jnp.float32),
            pltpu.VMEM((2, _SB, _LANES), jnp.float32),
            pltpu.VMEM((2, _SB, _LANES, _DIM), jnp.float32),
            pltpu.VMEM((2, _SB, _LANES, _DIM), jnp.float32),
            pltpu.VMEM((_ZROWS, _DIM), jnp.float32),
            [pltpu.SemaphoreType.DMA] * 2,
            [pltpu.SemaphoreType.DMA] * 2,
        ],
    )(_msgpass_body)
    return fn(table, src2d, dst2d, kv02d, kv12d)


def _merge_body(p_ref, c0_ref, c1_ref):
    c0_ref[...] = p_ref[0, 0] + p_ref[1, 0]
    c1_ref[...] = p_ref[0, 1] + p_ref[1, 1]


def _merge(p1):
    B = 2000
    out = jax.ShapeDtypeStruct((_N, _DIM), jnp.float32)
    return pl.pallas_call(
        _merge_body,
        grid=(_N // B,),
        in_specs=[pl.BlockSpec((_NC, 2, B, _DIM), lambda i: (0, 0, i, 0))],
        out_specs=[pl.BlockSpec((B, _DIM), lambda i: (i, 0))] * 2,
        out_shape=[out, out],
    )(p1)


def _mlp_body(h_ref, c0_ref, c1_ref, pa_ref, pb_ref,
              w1_ref, b1_ref, w2_ref, b2_ref, o_ref):
    a00 = pa_ref[0, 0] + pa_ref[1, 0]
    a01 = pb_ref[0, 0] + pb_ref[1, 0]
    a10 = pa_ref[0, 1] + pa_ref[1, 1]
    a11 = pb_ref[0, 1] + pb_ref[1, 1]
    feat = jnp.concatenate(
        [h_ref[...], c0_ref[...], c1_ref[...], a00, a01, a10, a11], axis=1)
    hid = jnp.dot(feat, w1_ref[...], preferred_element_type=jnp.float32)
    hid = jnp.maximum(hid + b1_ref[...][None, :], 0.0)
    o_ref[...] = (
        jnp.dot(hid, w2_ref[...], preferred_element_type=jnp.float32)
        + b2_ref[...][None, :])


def _mlp(h, c0, c1, p2a, p2b, W1, b1, W2, b2):
    B = 2000
    part = pl.BlockSpec((_NC, 2, B, _DIM), lambda i: (0, 0, i, 0))
    nd = pl.BlockSpec((B, _DIM), lambda i: (i, 0))
    return pl.pallas_call(
        _mlp_body,
        grid=(_N // B,),
        in_specs=[
            nd, nd, nd, part, part,
            pl.BlockSpec(W1.shape, lambda i: (0, 0)),
            pl.BlockSpec(b1.shape, lambda i: (0,)),
            pl.BlockSpec(W2.shape, lambda i: (0, 0)),
            pl.BlockSpec(b2.shape, lambda i: (0,)),
        ],
        out_specs=pl.BlockSpec((B, 32), lambda i: (i, 0)),
        out_shape=jax.ShapeDtypeStruct((_N, 32), jnp.float32),
    )(h, c0, c1, p2a, p2b, W1, b1, W2, b2)


def kernel(x, gauges, kernel_vals, edge_index, n_id, W1, b1, W2, b2):
    src = edge_index[0].astype(jnp.int32)
    dst = edge_index[1].astype(jnp.int32)
    kv0 = kernel_vals[0].astype(jnp.float32)
    kv1 = kernel_vals[1].astype(jnp.float32)
    pad = _EP - _E
    src2d = jnp.pad(src, (0, pad)).reshape(_R, _LANES)
    dst2d = jnp.pad(dst, (0, pad)).reshape(_R, _LANES)
    kv02d = jnp.pad(kv0, (0, pad)).reshape(_R, _LANES)
    kv12d = jnp.pad(kv1, (0, pad)).reshape(_R, _LANES)

    h = _prep(x, gauges)
    p1 = _msgpass(h, src2d, dst2d, kv02d, kv12d)
    c0, c1 = _merge(p1)
    p2a = _msgpass(c0, src2d, dst2d, kv02d, kv12d)
    p2b = _msgpass(c1, src2d, dst2d, kv02d, kv12d)
    return _mlp(h, c0, c1, p2a, p2b, W1, b1, W2, b2)


# speedup vs baseline: 17.2570x; 1.5625x over previous
import functools

import jax
import jax.numpy as jnp
from jax import lax
from jax.experimental import pallas as pl
from jax.experimental.pallas import tpu as pltpu
from jax.experimental.pallas import tpu_sc as plsc

_N = 50000
_DIM = 16
_E = 1600000
_NC = 2
_NS = 16
_NW = _NC * _NS
_LANES = 128
_BIGC = 8
_NBANK = 4
_RPW = 392
_R = _NW * _RPW
_EP = _R * _LANES
_NBIG = _RPW // _BIGC
_NP = 50048
_NPT = _NP // _NS


def _prep_body(x_ref, g_ref, e_ref, r_ref, h_ref):
    xb = jnp.dot(x_ref[...], e_ref[...], preferred_element_type=jnp.float32)
    prod = xb * g_ref[...]
    h = jnp.dot(prod, r_ref[...], preferred_element_type=jnp.float32)
    nrm = jnp.maximum(jnp.sqrt(jnp.sum(h * h, axis=1, keepdims=True)), 1e-12)
    h_ref[...] = h / nrm


def _prep(x, gauges):
    B = 2000
    g2 = gauges.reshape(_N, _DIM * _DIM)
    em = jnp.kron(jnp.eye(_DIM, dtype=jnp.float32),
                  jnp.ones((1, _DIM), dtype=jnp.float32))
    rm = jnp.kron(jnp.ones((_DIM, 1), dtype=jnp.float32),
                  jnp.eye(_DIM, dtype=jnp.float32))
    return pl.pallas_call(
        _prep_body,
        grid=(_N // B,),
        in_specs=[
            pl.BlockSpec((B, _DIM), lambda i: (i, 0)),
            pl.BlockSpec((B, _DIM * _DIM), lambda i: (i, 0)),
            pl.BlockSpec(em.shape, lambda i: (0, 0)),
            pl.BlockSpec(rm.shape, lambda i: (0, 0)),
        ],
        out_specs=pl.BlockSpec((B, _DIM), lambda i: (i, 0)),
        out_shape=jax.ShapeDtypeStruct((_N, _DIM), jnp.float32),
    )(x, g2, em, rm)


def _msgpass_body(table_h, src_h, dst_h, kv0_h, kv1_h, out_h,
                  acc0, acc1, srcb, dstb, kv0b, kv1b, rows, msg,
                  gsems, ssems, isem):
    c = lax.axis_index("c")
    s = lax.axis_index("s")
    wid = s * _NC + c

    def _zrow(i, _):
        msg[0, i] = jnp.zeros((_DIM,), jnp.float32)
        return 0

    lax.fori_loop(0, _LANES, _zrow, 0)
    nbase = s * _NPT

    def _zcp(i, _):
        pltpu.sync_copy(msg.at[0], acc0.at[pl.ds(nbase + i * _LANES, _LANES)])
        pltpu.sync_copy(msg.at[0], acc1.at[pl.ds(nbase + i * _LANES, _LANES)])
        return 0

    nfull = _NPT // _LANES
    lax.fori_loop(0, nfull, _zcp, 0)
    rem = _NPT - nfull * _LANES
    pltpu.sync_copy(msg.at[0, pl.ds(0, rem)],
                    acc0.at[pl.ds(nbase + nfull * _LANES, rem)])
    pltpu.sync_copy(msg.at[0, pl.ds(0, rem)],
                    acc1.at[pl.ds(nbase + nfull * _LANES, rem)])
    plsc.subcore_barrier()

    row0 = wid * _RPW
    idx_bufs = (srcb, dstb, kv0b, kv1b)
    idx_hbms = (src_h, dst_h, kv0_h, kv1_h)

    def _fire_idx(ib, r0):
        for hbm, buf in zip(idx_hbms, idx_bufs):
            pltpu.async_copy(hbm.at[pl.ds(r0, _BIGC)], buf.at[ib], isem)

    def _drain_idx(ib):
        for hbm, buf in zip(idx_hbms, idx_bufs):
            pltpu.make_async_copy(hbm.at[pl.ds(0, _BIGC)], buf.at[ib],
                                  isem).wait()

    def _fire_gather(ib, k, m):
        pltpu.async_copy(table_h.at[srcb.at[ib, k]], rows.at[m], gsems[m])

    def _drain_gather(m):
        pltpu.make_async_copy(table_h.at[pl.ds(0, _LANES)], rows.at[m],
                              gsems[m]).wait()

    def _drain_scatters(m):
        pltpu.make_async_copy(table_h.at[pl.ds(0, _LANES)], rows.at[m],
                              ssems[m]).wait()
        pltpu.make_async_copy(table_h.at[pl.ds(0, _LANES)], msg.at[m],
                              ssems[m]).wait()

    def _compute(ib, k, m):
        def _blk(b, _):
            kvec0 = kv0b[ib, k, pl.ds(b * _DIM, _DIM)]
            kvec1 = kv1b[ib, k, pl.ds(b * _DIM, _DIM)]
            for ee in range(_DIM):
                e = b * _DIM + ee
                v = rows[m, e]
                msg[m, e] = v * kvec1[ee]
                rows[m, e] = v * kvec0[ee]
            return 0

        lax.fori_loop(0, _LANES // _DIM, _blk, 0)
        pltpu.async_copy(rows.at[m], acc0.at[dstb.at[ib, k]], ssems[m],
                         add=True)
        pltpu.async_copy(msg.at[m], acc1.at[dstb.at[ib, k]], ssems[m],
                         add=True)

    def _chunk(ib, k, B, first_B):
        m = k % _NBANK
        nx = (k + 1) % _NBANK
        if not (first_B and k < 3):
            _drain_scatters(nx)
        if k == 3 and not first_B:
            @pl.when(B < _NBIG - 1)
            def _():
                _fire_idx(1 - ib, row0 + (B + 1) * _BIGC)
        if k < _BIGC - 1:
            _fire_gather(ib, k + 1, nx)
        else:
            if first_B:
                _drain_idx(1 - ib)
                _fire_gather(1 - ib, 0, nx)
            else:
                @pl.when(B < _NBIG - 1)
                def _():
                    _drain_idx(1 - ib)
                    _fire_gather(1 - ib, 0, nx)
        _drain_gather(m)
        _compute(ib, k, m)

    for hbm, buf in zip(idx_hbms, idx_bufs):
        pltpu.sync_copy(hbm.at[pl.ds(row0, _BIGC)], buf.at[0])
    _fire_idx(1, row0 + _BIGC)
    _fire_gather(0, 0, 0)
    for k in range(_BIGC):
        _chunk(0, k, 0, True)

    def _pair(t, _):
        for bb in range(2):
            B = 2 * t + 1 + bb
            ib = (1 + bb) % 2
            for k in range(_BIGC):
                _chunk(ib, k, B, False)
        return 0

    lax.fori_loop(0, (_NBIG - 1) // 2, _pair, 0)

    for m in (1, 2, 3):
        _drain_scatters(m)
    plsc.subcore_barrier()

    pltpu.sync_copy(acc0.at[pl.ds(nbase, _NPT)],
                    out_h.at[c, 0, pl.ds(nbase, _NPT)])
    pltpu.sync_copy(acc1.at[pl.ds(nbase, _NPT)],
                    out_h.at[c, 1, pl.ds(nbase, _NPT)])


def _msgpass(table, src2d, dst2d, kv02d, kv12d):
    mesh = plsc.VectorSubcoreMesh(
        core_axis_name="c", subcore_axis_name="s",
        num_cores=_NC, num_subcores=_NS,
    )
    fn = functools.partial(
        pl.kernel,
        out_type=jax.ShapeDtypeStruct((_NC, 2, _NP, _DIM), jnp.float32),
        mesh=mesh,
        compiler_params=pltpu.CompilerParams(use_tc_tiling_on_sc=False),
        scratch_types=[
            pltpu.VMEM_SHARED((_NP, _DIM), jnp.float32),
            pltpu.VMEM_SHARED((_NP, _DIM), jnp.float32),
            pltpu.VMEM((2, _BIGC, _LANES), jnp.int32),
            pltpu.VMEM((2, _BIGC, _LANES), jnp.int32),
            pltpu.VMEM((2, _BIGC, _LANES), jnp.float32),
            pltpu.VMEM((2, _BIGC, _LANES), jnp.float32),
            pltpu.VMEM((_NBANK, _LANES, _DIM), jnp.float32),
            pltpu.VMEM((_NBANK, _LANES, _DIM), jnp.float32),
            [pltpu.SemaphoreType.DMA] * _NBANK,
            [pltpu.SemaphoreType.DMA] * _NBANK,
            pltpu.SemaphoreType.DMA,
        ],
    )(_msgpass_body)
    return fn(table, src2d, dst2d, kv02d, kv12d)


def _merge_body(p_ref, c0_ref, c1_ref):
    c0_ref[...] = p_ref[0, 0] + p_ref[1, 0]
    c1_ref[...] = p_ref[0, 1] + p_ref[1, 1]


def _merge(p1):
    B = 2000
    out = jax.ShapeDtypeStruct((_N, _DIM), jnp.float32)
    return pl.pallas_call(
        _merge_body,
        grid=(_N // B,),
        in_specs=[pl.BlockSpec((_NC, 2, B, _DIM), lambda i: (0, 0, i, 0))],
        out_specs=[pl.BlockSpec((B, _DIM), lambda i: (i, 0))] * 2,
        out_shape=[out, out],
    )(p1)


def _mlp_body(h_ref, c0_ref, c1_ref, pa_ref, pb_ref,
              w1_ref, b1_ref, w2_ref, b2_ref, o_ref):
    a00 = pa_ref[0, 0] + pa_ref[1, 0]
    a01 = pb_ref[0, 0] + pb_ref[1, 0]
    a10 = pa_ref[0, 1] + pa_ref[1, 1]
    a11 = pb_ref[0, 1] + pb_ref[1, 1]
    feat = jnp.concatenate(
        [h_ref[...], c0_ref[...], c1_ref[...], a00, a01, a10, a11], axis=1)
    hid = jnp.dot(feat, w1_ref[...], preferred_element_type=jnp.float32)
    hid = jnp.maximum(hid + b1_ref[...][None, :], 0.0)
    o_ref[...] = (
        jnp.dot(hid, w2_ref[...], preferred_element_type=jnp.float32)
        + b2_ref[...][None, :])


def _mlp(h, c0, c1, p2a, p2b, W1, b1, W2, b2):
    B = 2000
    part = pl.BlockSpec((_NC, 2, B, _DIM), lambda i: (0, 0, i, 0))
    nd = pl.BlockSpec((B, _DIM), lambda i: (i, 0))
    return pl.pallas_call(
        _mlp_body,
        grid=(_N // B,),
        in_specs=[
            nd, nd, nd, part, part,
            pl.BlockSpec(W1.shape, lambda i: (0, 0)),
            pl.BlockSpec(b1.shape, lambda i: (0,)),
            pl.BlockSpec(W2.shape, lambda i: (0, 0)),
            pl.BlockSpec(b2.shape, lambda i: (0,)),
        ],
        out_specs=pl.BlockSpec((B, 32), lambda i: (i, 0)),
        out_shape=jax.ShapeDtypeStruct((_N, 32), jnp.float32),
    )(h, c0, c1, p2a, p2b, W1, b1, W2, b2)


def kernel(x, gauges, kernel_vals, edge_index, n_id, W1, b1, W2, b2):
    src = edge_index[0].astype(jnp.int32)
    dst = edge_index[1].astype(jnp.int32)
    kv0 = kernel_vals[0].astype(jnp.float32)
    kv1 = kernel_vals[1].astype(jnp.float32)
    pad = _EP - _E
    src2d = jnp.pad(src, (0, pad)).reshape(_R, _LANES)
    dst2d = jnp.pad(dst, (0, pad)).reshape(_R, _LANES)
    kv02d = jnp.pad(kv0, (0, pad)).reshape(_R, _LANES)
    kv12d = jnp.pad(kv1, (0, pad)).reshape(_R, _LANES)

    h = _prep(x, gauges)
    p1 = _msgpass(h, src2d, dst2d, kv02d, kv12d)
    c0, c1 = _merge(p1)
    p2a = _msgpass(c0, src2d, dst2d, kv02d, kv12d)
    p2b = _msgpass(c1, src2d, dst2d, kv02d, kv12d)
    return _mlp(h, c0, c1, p2a, p2b, W1, b1, W2, b2)


# speedup vs baseline: 22.5733x; 1.3081x over previous
import functools

import jax
import jax.numpy as jnp
from jax import lax
from jax.experimental import pallas as pl
from jax.experimental.pallas import tpu as pltpu
from jax.experimental.pallas import tpu_sc as plsc

_N = 50000
_DIM = 16
_E = 1600000
_NC = 2
_NS = 16
_NW = _NC * _NS
_LANES = 128
_BIGC = 8
_NBANK = 4
_RPW = 392
_R = _NW * _RPW
_EP = _R * _LANES
_NP = 50048
_NPT = _NP // _NS


def _prep_body(x_ref, g_ref, e_ref, r_ref, h_ref):
    xb = jnp.dot(x_ref[...], e_ref[...], preferred_element_type=jnp.float32)
    prod = xb * g_ref[...]
    h = jnp.dot(prod, r_ref[...], preferred_element_type=jnp.float32)
    nrm = jnp.maximum(jnp.sqrt(jnp.sum(h * h, axis=1, keepdims=True)), 1e-12)
    h_ref[...] = h / nrm


def _prep(x, gauges):
    B = 2000
    g2 = gauges.reshape(_N, _DIM * _DIM)
    em = jnp.kron(jnp.eye(_DIM, dtype=jnp.float32),
                  jnp.ones((1, _DIM), dtype=jnp.float32))
    rm = jnp.kron(jnp.ones((_DIM, 1), dtype=jnp.float32),
                  jnp.eye(_DIM, dtype=jnp.float32))
    return pl.pallas_call(
        _prep_body,
        grid=(_N // B,),
        in_specs=[
            pl.BlockSpec((B, _DIM), lambda i: (i, 0)),
            pl.BlockSpec((B, _DIM * _DIM), lambda i: (i, 0)),
            pl.BlockSpec(em.shape, lambda i: (0, 0)),
            pl.BlockSpec(rm.shape, lambda i: (0, 0)),
        ],
        out_specs=pl.BlockSpec((B, _DIM), lambda i: (i, 0)),
        out_shape=jax.ShapeDtypeStruct((_N, _DIM), jnp.float32),
    )(x, g2, em, rm)


def _make_msgpass_body(core_split):
    rpw = _RPW if not core_split else _RPW * _NC
    nbig = rpw // _BIGC

    def body(table_h, ed_h, out_h, acc0, acc1, edata, rm, gsems, ssems, isem):
        cc = lax.axis_index("c")
        s = lax.axis_index("s")

        def _zrow(i, _):
            rm[0, i] = jnp.zeros((_DIM,), jnp.float32)
            return 0

        lax.fori_loop(0, _LANES, _zrow, 0)
        nbase = s * _NPT

        def _zcp(i, _):
            pltpu.sync_copy(rm.at[0, pl.ds(0, _LANES)],
                            acc0.at[pl.ds(nbase + i * _LANES, _LANES)])
            pltpu.sync_copy(rm.at[0, pl.ds(0, _LANES)],
                            acc1.at[pl.ds(nbase + i * _LANES, _LANES)])
            return 0

        nfull = _NPT // _LANES
        lax.fori_loop(0, nfull, _zcp, 0)
        rem = _NPT - nfull * _LANES
        pltpu.sync_copy(rm.at[0, pl.ds(0, rem)],
                        acc0.at[pl.ds(nbase + nfull * _LANES, rem)])
        pltpu.sync_copy(rm.at[0, pl.ds(0, rem)],
                        acc1.at[pl.ds(nbase + nfull * _LANES, rem)])
        plsc.subcore_barrier()

        if core_split:
            row0 = s * rpw
            goff = cc * _N
        else:
            row0 = (s * _NC + cc) * rpw
            goff = None

        def _add_goff(ib):
            for r in range(_BIGC):
                for j in range(_LANES // _DIM):
                    sl = pl.ds(j * _DIM, _DIM)
                    edata[ib, r, 0, sl] = edata[ib, r, 0, sl] + goff

        def _fire_idx(ib, r0):
            pltpu.async_copy(ed_h.at[pl.ds(r0, _BIGC)], edata.at[ib], isem)

        def _drain_idx(ib):
            pltpu.make_async_copy(ed_h.at[pl.ds(0, _BIGC)], edata.at[ib],
                                  isem).wait()
            if core_split:
                _add_goff(ib)

        def _fire_gather(ib, k, m):
            pltpu.async_copy(table_h.at[edata.at[ib, k, 0]],
                             rm.at[m, pl.ds(0, _LANES)], gsems[m])

        def _drain_gather(m):
            pltpu.make_async_copy(table_h.at[pl.ds(0, _LANES)],
                                  rm.at[m, pl.ds(0, _LANES)], gsems[m]).wait()

        def _drain_scatters(m):
            pltpu.make_async_copy(table_h.at[pl.ds(0, 2 * _LANES)], rm.at[m],
                                  ssems[m]).wait()

        def _compute(ib, k, m):
            def _blk(b, _):
                kvec0 = plsc.bitcast(
                    edata[ib, k, 2, pl.ds(b * _DIM, _DIM)], jnp.float32)
                kvec1 = plsc.bitcast(
                    edata[ib, k, 3, pl.ds(b * _DIM, _DIM)], jnp.float32)
                for ee in range(_DIM):
                    e = b * _DIM + ee
                    v = rm[m, e]
                    rm[m, _LANES + e] = v * kvec1[ee]
                    rm[m, e] = v * kvec0[ee]
                return 0

            lax.fori_loop(0, _LANES // _DIM, _blk, 0)
            pltpu.async_copy(rm.at[m, pl.ds(0, _LANES)],
                             acc0.at[edata.at[ib, k, 1]], ssems[m], add=True)
            pltpu.async_copy(rm.at[m, pl.ds(_LANES, _LANES)],
                             acc1.at[edata.at[ib, k, 1]], ssems[m], add=True)

        def _chunk(ib, k, B):
            static = isinstance(B, int)
            m = k % _NBANK
            nx = (k + 1) % _NBANK
            if not (static and B == 0 and k < 3):
                _drain_scatters(nx)
            if k == 3 and not (static and B == 0):
                if static:
                    if B < nbig - 1:
                        _fire_idx(1 - ib, row0 + (B + 1) * _BIGC)
                else:
                    @pl.when(B < nbig - 1)
                    def _():
                        _fire_idx(1 - ib, row0 + (B + 1) * _BIGC)
            if k < _BIGC - 1:
                _fire_gather(ib, k + 1, nx)
            else:
                if static:
                    if B < nbig - 1:
                        _drain_idx(1 - ib)
                        _fire_gather(1 - ib, 0, nx)
                else:
                    @pl.when(B < nbig - 1)
                    def _():
                        _drain_idx(1 - ib)
                        _fire_gather(1 - ib, 0, nx)
            _drain_gather(m)
            _compute(ib, k, m)

        pltpu.sync_copy(ed_h.at[pl.ds(row0, _BIGC)], edata.at[0])
        if core_split:
            _add_goff(0)
        _fire_idx(1, row0 + _BIGC)
        _fire_gather(0, 0, 0)
        for k in range(_BIGC):
            _chunk(0, k, 0)
        start = 1
        if (nbig - start) % 2 != 0:
            for k in range(_BIGC):
                _chunk(1, k, 1)
            start = 2

        def _pair(t, _):
            for bb in range(2):
                B = 2 * t + start + bb
                ib = (start + bb) % 2
                for k in range(_BIGC):
                    _chunk(ib, k, B)
            return 0

        lax.fori_loop(0, (nbig - start) // 2, _pair, 0)

        for m in (1, 2, 3):
            _drain_scatters(m)
        plsc.subcore_barrier()

        pltpu.sync_copy(acc0.at[pl.ds(nbase, _NPT)],
                        out_h.at[cc, 0, pl.ds(nbase, _NPT)])
        pltpu.sync_copy(acc1.at[pl.ds(nbase, _NPT)],
                        out_h.at[cc, 1, pl.ds(nbase, _NPT)])

    return body


def _msgpass(table, edata, core_split):
    mesh = plsc.VectorSubcoreMesh(
        core_axis_name="c", subcore_axis_name="s",
        num_cores=_NC, num_subcores=_NS,
    )
    fn = functools.partial(
        pl.kernel,
        out_type=jax.ShapeDtypeStruct((_NC, 2, _NP, _DIM), jnp.float32),
        mesh=mesh,
        compiler_params=pltpu.CompilerParams(
            use_tc_tiling_on_sc=False, needs_layout_passes=False),
        scratch_types=[
            pltpu.VMEM_SHARED((_NP, _DIM), jnp.float32),
            pltpu.VMEM_SHARED((_NP, _DIM), jnp.float32),
            pltpu.VMEM((2, _BIGC, 4, _LANES), jnp.int32),
            pltpu.VMEM((_NBANK, 2 * _LANES, _DIM), jnp.float32),
            [pltpu.SemaphoreType.DMA] * _NBANK,
            [pltpu.SemaphoreType.DMA] * _NBANK,
            pltpu.SemaphoreType.DMA,
        ],
    )(_make_msgpass_body(core_split))
    return fn(table, edata)


def _merge_body(p_ref, c0_ref, c1_ref):
    c0_ref[...] = p_ref[0, 0] + p_ref[1, 0]
    c1_ref[...] = p_ref[0, 1] + p_ref[1, 1]


def _merge(p1):
    B = 2000
    out = jax.ShapeDtypeStruct((_N, _DIM), jnp.float32)
    return pl.pallas_call(
        _merge_body,
        grid=(_N // B,),
        in_specs=[pl.BlockSpec((_NC, 2, B, _DIM), lambda i: (0, 0, i, 0))],
        out_specs=[pl.BlockSpec((B, _DIM), lambda i: (i, 0))] * 2,
        out_shape=[out, out],
    )(p1)


def _mlp_body(h_ref, c0_ref, c1_ref, p2_ref,
              w1_ref, b1_ref, w2_ref, b2_ref, o_ref):
    feat = jnp.concatenate(
        [h_ref[...], c0_ref[...], c1_ref[...],
         p2_ref[0, 0], p2_ref[1, 0], p2_ref[0, 1], p2_ref[1, 1]], axis=1)
    hid = jnp.dot(feat, w1_ref[...], preferred_element_type=jnp.float32)
    hid = jnp.maximum(hid + b1_ref[...][None, :], 0.0)
    o_ref[...] = (
        jnp.dot(hid, w2_ref[...], preferred_element_type=jnp.float32)
        + b2_ref[...][None, :])


def _mlp(h, c0, c1, p2, W1, b1, W2, b2):
    B = 2000
    part = pl.BlockSpec((_NC, 2, B, _DIM), lambda i: (0, 0, i, 0))
    nd = pl.BlockSpec((B, _DIM), lambda i: (i, 0))
    return pl.pallas_call(
        _mlp_body,
        grid=(_N // B,),
        in_specs=[
            nd, nd, nd, part,
            pl.BlockSpec(W1.shape, lambda i: (0, 0)),
            pl.BlockSpec(b1.shape, lambda i: (0,)),
            pl.BlockSpec(W2.shape, lambda i: (0, 0)),
            pl.BlockSpec(b2.shape, lambda i: (0,)),
        ],
        out_specs=pl.BlockSpec((B, 32), lambda i: (i, 0)),
        out_shape=jax.ShapeDtypeStruct((_N, 32), jnp.float32),
    )(h, c0, c1, p2, W1, b1, W2, b2)


def kernel(x, gauges, kernel_vals, edge_index, n_id, W1, b1, W2, b2):
    src = edge_index[0].astype(jnp.int32)
    dst = edge_index[1].astype(jnp.int32)
    kv0 = lax.bitcast_convert_type(
        kernel_vals[0].astype(jnp.float32), jnp.int32)
    kv1 = lax.bitcast_convert_type(
        kernel_vals[1].astype(jnp.float32), jnp.int32)
    pad = _EP - _E

    def _pad2d(a):
        return jnp.pad(a, (0, pad)).reshape(_R, _LANES)

    edata = jnp.stack([_pad2d(src), _pad2d(dst), _pad2d(kv0), _pad2d(kv1)],
                      axis=1)

    h = _prep(x, gauges)
    p1 = _msgpass(h, edata, core_split=False)
    c0, c1 = _merge(p1)
    tbl2 = jnp.concatenate([c0, c1], axis=0)
    p2 = _msgpass(tbl2, edata, core_split=True)
    return _mlp(h, c0, c1, p2, W1, b1, W2, b2)
